# Initial kernel scaffold; baseline (speedup 1.0000x reference)
import jax
import jax.numpy as jnp
from jax.experimental import pallas as pl


def kernel(curr_pos, curr_motion, prev_motion, edge_index0, edge_index1, edge_index2, edge_index3, down0, down1, down2, up0, up1, up2, params):
    raise NotImplementedError("write your pallas kernel here")


# speedup vs baseline: 8.3544x; 8.3544x over previous
import functools
import math

import jax
import jax.numpy as jnp
from jax import lax
from jax.experimental import pallas as pl
from jax.experimental.pallas import tpu as pltpu
from jax.experimental.pallas import tpu_sc as plsc

HID = 32
NC = 2
NS = 16
NW = NC * NS
LANE = 16


def _rup(x, m):
    return (x + m - 1) // m * m


def _lstm_enc_body(pm_ref, cp_ref, cm_ref, wih0_ref, whh0_ref, b0_ref,
                   wih1_ref, whh1_ref, b1_ref, seqw_ref, seqb_ref,
                   encw_ref, encb_ref, out_ref):
    T = pm_ref.shape[0]
    B = cp_ref.shape[0]
    h0 = jnp.zeros((B, HID), jnp.float32)
    c0 = jnp.zeros((B, HID), jnp.float32)
    h1 = jnp.zeros((B, HID), jnp.float32)
    c1 = jnp.zeros((B, HID), jnp.float32)

    def cell(xt, h, c, wih, whh, b):
        g = (jnp.dot(xt, wih, preferred_element_type=jnp.float32)
             + jnp.dot(h, whh, preferred_element_type=jnp.float32) + b)
        i = g[:, :HID]
        f = g[:, HID:2 * HID]
        gg = g[:, 2 * HID:3 * HID]
        o = g[:, 3 * HID:]
        i = 1.0 / (1.0 + jnp.exp(-i))
        f = 1.0 / (1.0 + jnp.exp(-f))
        gg = jnp.tanh(gg)
        o = 1.0 / (1.0 + jnp.exp(-o))
        c = f * c + i * gg
        h = o * jnp.tanh(c)
        return h, c

    for t in range(T):
        xt = pm_ref[t]
        h0, c0 = cell(xt, h0, c0, wih0_ref[...], whh0_ref[...], b0_ref[...])
        h1, c1 = cell(h0, h1, c1, wih1_ref[...], whh1_ref[...], b1_ref[...])
    seq_pred = jnp.dot(h1, seqw_ref[...], preferred_element_type=jnp.float32) + seqb_ref[...]
    encw = encw_ref[...]
    x = (jnp.dot(cp_ref[...], encw[:3], preferred_element_type=jnp.float32)
         + jnp.dot(seq_pred, encw[3:7], preferred_element_type=jnp.float32)
         + jnp.dot(cm_ref[...], encw[7:], preferred_element_type=jnp.float32)
         + encb_ref[...])
    out_ref[...] = x


def _lstm_enc(pm, cp, cm, p, n_pad, bl):
    T = pm.shape[0]
    grid = n_pad // bl
    full = lambda a: pl.BlockSpec(a.shape, lambda i: tuple(0 for _ in a.shape))
    l0, l1 = p['lstm'][0], p['lstm'][1]
    args = [
        pm, cp, cm,
        l0['Wih'].T, l0['Whh'].T, (l0['bih'] + l0['bhh']).reshape(1, -1),
        l1['Wih'].T, l1['Whh'].T, (l1['bih'] + l1['bhh']).reshape(1, -1),
        p['seq_W'], p['seq_b'].reshape(1, -1),
        p['enc_W'], p['enc_b'].reshape(1, -1),
    ]
    specs = [
        pl.BlockSpec((T, bl, 4), lambda i: (0, i, 0)),
        pl.BlockSpec((bl, 3), lambda i: (i, 0)),
        pl.BlockSpec((bl, 4), lambda i: (i, 0)),
    ] + [full(a) for a in args[3:]]
    return pl.pallas_call(
        _lstm_enc_body,
        grid=(grid,),
        in_specs=specs,
        out_specs=pl.BlockSpec((bl, HID), lambda i: (i, 0)),
        out_shape=jax.ShapeDtypeStruct((n_pad, HID), jnp.float32),
    )(*args)


def _proj_body(x_ref, g_ref, b_ref, w_ref, bias_ref, out_ref, *, ln):
    x = x_ref[...]
    if ln:
        m = jnp.mean(x, -1, keepdims=True)
        v = jnp.mean((x - m) ** 2, -1, keepdims=True)
        x = (x - m) / jnp.sqrt(v + 1e-5) * g_ref[...] + b_ref[...]
        x = jnp.maximum(x, 0.0)
    out_ref[...] = jnp.dot(x, w_ref[...], preferred_element_type=jnp.float32) + bias_ref[...]


def _proj(x, gam, bet, conv, n_pad, bl, ln):
    C = x.shape[1]
    wcat = jnp.concatenate([conv['Wq'], conv['Wk'], conv['Wv'], conv['Ws']], axis=1)
    bcat = jnp.concatenate([conv['bq'], conv['bk'], conv['bv'], conv['bs']]).reshape(1, -1)
    grid = n_pad // bl
    full = lambda a: pl.BlockSpec(a.shape, lambda i: tuple(0 for _ in a.shape))
    args = [x, gam.reshape(1, -1), bet.reshape(1, -1), wcat, bcat]
    specs = [pl.BlockSpec((bl, C), lambda i: (i, 0))] + [full(a) for a in args[1:]]
    return pl.pallas_call(
        functools.partial(_proj_body, ln=ln),
        grid=(grid,),
        in_specs=specs,
        out_specs=pl.BlockSpec((bl, 4 * C), lambda i: (i, 0)),
        out_shape=jax.ShapeDtypeStruct((n_pad, 4 * C), jnp.float32),
    )(*args)


def _fin_body(*refs, n_agg, resid):
    out_ref = refs[-1]
    x4 = refs[0][...]
    C = x4.shape[1] // 4
    sproj = x4[:, 3 * C:]
    s_ref = refs[1]
    aggs = refs[2:2 + n_agg]
    idx = 2 + n_agg
    s2 = s_ref[...]
    ssum = s2[0] + s2[1] + 1e-16
    parts = []
    for a in aggs:
        a2 = a[...]
        parts.append(a2[0] + a2[1])
    agg = jnp.concatenate(parts, axis=-1) if n_agg > 1 else parts[0]
    out = sproj + agg / ssum
    if resid:
        out = out + refs[idx][...]
    out_ref[...] = out


def _finalize(out4, s_part, agg_parts, xin, n_pad, bl, C):
    grid = n_pad // bl
    n_agg = len(agg_parts)
    cc = agg_parts[0].shape[2]
    s3 = s_part.reshape(2, n_pad, 1)
    args = [out4, s3] + list(agg_parts)
    specs = [
        pl.BlockSpec((bl, 4 * C), lambda i: (i, 0)),
        pl.BlockSpec((2, bl, 1), lambda i: (0, i, 0)),
    ] + [pl.BlockSpec((2, bl, cc), lambda i: (0, i, 0)) for _ in agg_parts]
    if xin is not None:
        args.append(xin)
        specs.append(pl.BlockSpec((bl, C), lambda i: (i, 0)))
    return pl.pallas_call(
        functools.partial(_fin_body, n_agg=n_agg, resid=xin is not None),
        grid=(grid,),
        in_specs=specs,
        out_specs=pl.BlockSpec((bl, C), lambda i: (i, 0)),
        out_shape=jax.ShapeDtypeStruct((n_pad, C), jnp.float32),
    )(*args)


def _head_body(x_ref, g_ref, b_ref, w_ref, bias_ref, out_ref):
    x = x_ref[...]
    m = jnp.mean(x, -1, keepdims=True)
    v = jnp.mean((x - m) ** 2, -1, keepdims=True)
    x = (x - m) / jnp.sqrt(v + 1e-5) * g_ref[...] + b_ref[...]
    x = jnp.maximum(x, 0.0)
    pred = jnp.dot(x, w_ref[...], preferred_element_type=jnp.float32) + bias_ref[...]
    lane = lax.broadcasted_iota(jnp.int32, pred.shape, 1)
    sp = jnp.where(pred > 20.0, pred, jnp.log1p(jnp.exp(jnp.minimum(pred, 20.0))))
    out_ref[...] = jnp.where(lane == 3, sp, pred)


def _head(x, p, n_pad, bl):
    C = x.shape[1]
    w8 = jnp.concatenate([p['lin_W'], jnp.zeros((C, 4), jnp.float32)], axis=1)
    b8 = jnp.concatenate([p['lin_b'], jnp.zeros((4,), jnp.float32)]).reshape(1, -1)
    grid = n_pad // bl
    full = lambda a: pl.BlockSpec(a.shape, lambda i: tuple(0 for _ in a.shape))
    args = [x, p['norm_g'].reshape(1, -1), p['norm_b'].reshape(1, -1), w8, b8]
    specs = [pl.BlockSpec((bl, C), lambda i: (i, 0))] + [full(a) for a in args[1:]]
    return pl.pallas_call(
        _head_body,
        grid=(grid,),
        in_specs=specs,
        out_specs=pl.BlockSpec((bl, 8), lambda i: (i, 0)),
        out_shape=jax.ShapeDtypeStruct((n_pad, 8), jnp.float32),
    )(*args)


def _zdiv(rps, cap):
    for d in range(min(rps, cap) // 16 * 16, 0, -16):
        if rps % d == 0:
            return d
    return 8


def _zero16():
    return jnp.zeros((LANE,), jnp.float32)


def _hsum16(acc, idx16):
    for step in (8, 4, 2, 1):
        acc = acc + jnp.take(acc, idx16 ^ step)
    return acc


def _attn_fused_body(qkv, srcr, dstr, s_out, agg_out,
                     sbuf, dbuf, qib, kib, vib, qbuf, kbuf, vbuf, wbuf,
                     zrow, zs1, s_sh, agg_sh, semq, semk, semv,
                     *, n_pad, C, e_pad, be, zc):
    cid = lax.axis_index("c")
    sid = lax.axis_index("s")
    wid = sid * NC + cid
    rps = n_pad // NS
    r0 = sid * rps
    z16 = _zero16()

    def zr(r, _):
        for c0 in range(0, C, 16):
            zrow[r, pl.ds(c0, 16)] = z16
        return 0
    lax.fori_loop(0, zc, zr, 0)

    def zs_(r, _):
        zs1[pl.ds(r * 16, 16)] = z16
        return 0
    lax.fori_loop(0, zc // 16, zs_, 0)

    def zcpy(i, _):
        rr = r0 + i * zc
        pltpu.sync_copy(zrow, agg_sh.at[pl.ds(rr, zc)])
        pltpu.sync_copy(zs1, s_sh.at[pl.ds(rr, zc)])
        return 0
    lax.fori_loop(0, rps // zc, zcpy, 0)
    plsc.subcore_barrier()
    epw = e_pad // NW
    base = wid * epw
    sub = be // 128
    inv = 1.0 / math.sqrt(C)
    iota = lax.iota(jnp.int32, LANE)

    def blk(j, _):
        off = base + j * be

        def prep(j2, _):
            pltpu.sync_copy(srcr.at[pl.ds(off + j2 * 128, 128)], sbuf.at[j2])
            pltpu.sync_copy(dstr.at[pl.ds(off + j2 * 128, 128)], dbuf.at[j2])
            for gg in range(8):
                sl = pl.ds(gg * 16, 16)
                sv = sbuf[j2, sl] * 4
                dv = dbuf[j2, sl] * 4
                qib[j2, sl] = dv
                kib[j2, sl] = sv + 1
                vib[j2, sl] = sv + 2
            return 0
        lax.fori_loop(0, sub, prep, 0)
        cps = []
        for j2 in range(sub):
            cps.append(pltpu.async_copy(qkv.at[qib.at[j2]], qbuf.at[pl.ds(j2 * 128, 128)], semq))
            cps.append(pltpu.async_copy(qkv.at[kib.at[j2]], kbuf.at[pl.ds(j2 * 128, 128)], semk))
            cps.append(pltpu.async_copy(qkv.at[vib.at[j2]], vbuf.at[pl.ds(j2 * 128, 128)], semv))
        for c in cps:
            c.wait()

        def score(g, _):
            svec = jnp.zeros((LANE,), jnp.float32)
            for ii in range(LANE):
                e = g * 16 + ii
                acc = qbuf[e, pl.ds(0, 16)] * kbuf[e, pl.ds(0, 16)]
                for c0 in range(16, C, 16):
                    acc = acc + qbuf[e, pl.ds(c0, 16)] * kbuf[e, pl.ds(c0, 16)]
                svec = jnp.where(iota == ii, _hsum16(acc, iota)[ii], svec)
            w = jnp.exp(svec * inv)
            wbuf[pl.ds(g * 16, 16)] = w
            for ii in range(LANE):
                e = g * 16 + ii
                we = w[ii]
                for c0 in range(0, C, 16):
                    vbuf[e, pl.ds(c0, 16)] = we * vbuf[e, pl.ds(c0, 16)]
            return 0
        lax.fori_loop(0, be // 16, score, 0)
        for j2 in range(sub):
            pltpu.sync_copy(wbuf.at[pl.ds(j2 * 128, 128)], s_sh.at[dbuf.at[j2]], add=True)
            pltpu.sync_copy(vbuf.at[pl.ds(j2 * 128, 128)], agg_sh.at[dbuf.at[j2]], add=True)
        return 0
    lax.fori_loop(0, epw // be, blk, 0)
    plsc.subcore_barrier()

    def outl(i, _):
        rr = r0 + i * zc
        pltpu.sync_copy(s_sh.at[pl.ds(rr, zc)], zs1)
        pltpu.sync_copy(zs1, s_out.at[pl.ds(cid * n_pad + rr, zc)])
        pltpu.sync_copy(agg_sh.at[pl.ds(rr, zc)], zrow)
        pltpu.sync_copy(zrow, agg_out.at[cid, pl.ds(rr, zc)])
        return 0
    lax.fori_loop(0, rps // zc, outl, 0)


def _attn_fused(qkv_view, src, dst, n_pad, C, e_pad, be):
    sub = be // 128
    zc = _zdiv(n_pad // NS, 32)
    kern = pl.kernel(
        functools.partial(_attn_fused_body, n_pad=n_pad, C=C, e_pad=e_pad, be=be, zc=zc),
        out_type=(jax.ShapeDtypeStruct((2 * n_pad,), jnp.float32),
                  jax.ShapeDtypeStruct((2, n_pad, C), jnp.float32)),
        mesh=plsc.VectorSubcoreMesh(core_axis_name="c", subcore_axis_name="s"),
        compiler_params=pltpu.CompilerParams(use_tc_tiling_on_sc=False),
        scratch_types=[
            pltpu.VMEM((sub, 128), jnp.int32),
            pltpu.VMEM((sub, 128), jnp.int32),
            pltpu.VMEM((sub, 128), jnp.int32),
            pltpu.VMEM((sub, 128), jnp.int32),
            pltpu.VMEM((sub, 128), jnp.int32),
            pltpu.VMEM((be, C), jnp.float32),
            pltpu.VMEM((be, C), jnp.float32),
            pltpu.VMEM((be, C), jnp.float32),
            pltpu.VMEM((be,), jnp.float32),
            pltpu.VMEM((zc, C), jnp.float32),
            pltpu.VMEM((zc,), jnp.float32),
            pltpu.VMEM_SHARED((n_pad,), jnp.float32),
            pltpu.VMEM_SHARED((n_pad, C), jnp.float32),
            pltpu.SemaphoreType.DMA,
            pltpu.SemaphoreType.DMA,
            pltpu.SemaphoreType.DMA,
        ],
    )
    return kern(qkv_view, src, dst)


def _attn_score_body(qkv, srcr, dstr, w_out, s_out,
                     sbuf, dbuf, qib, kib, qbuf, kbuf, wbuf, zs1,
                     s_sh, semq, semk,
                     *, n_pad, C, e_pad, be, zc):
    cid = lax.axis_index("c")
    sid = lax.axis_index("s")
    wid = sid * NC + cid
    rps = n_pad // NS
    r0 = sid * rps
    z16 = _zero16()

    def zs_(r, _):
        zs1[pl.ds(r * 16, 16)] = z16
        return 0
    lax.fori_loop(0, zc // 16, zs_, 0)

    def zcpy(i, _):
        pltpu.sync_copy(zs1, s_sh.at[pl.ds(r0 + i * zc, zc)])
        return 0
    lax.fori_loop(0, rps // zc, zcpy, 0)
    plsc.subcore_barrier()
    epw = e_pad // NW
    base = wid * epw
    sub = be // 128
    inv = 1.0 / math.sqrt(C)
    iota = lax.iota(jnp.int32, LANE)

    def blk(j, _):
        off = base + j * be

        def prep(j2, _):
            pltpu.sync_copy(srcr.at[pl.ds(off + j2 * 128, 128)], sbuf.at[j2])
            pltpu.sync_copy(dstr.at[pl.ds(off + j2 * 128, 128)], dbuf.at[j2])
            for gg in range(8):
                sl = pl.ds(gg * 16, 16)
                qib[j2, sl] = dbuf[j2, sl] * 4
                kib[j2, sl] = sbuf[j2, sl] * 4 + 1
            return 0
        lax.fori_loop(0, sub, prep, 0)
        cps = []
        for j2 in range(sub):
            cps.append(pltpu.async_copy(qkv.at[qib.at[j2]], qbuf.at[pl.ds(j2 * 128, 128)], semq))
            cps.append(pltpu.async_copy(qkv.at[kib.at[j2]], kbuf.at[pl.ds(j2 * 128, 128)], semk))
        for c in cps:
            c.wait()

        def score(g, _):
            svec = jnp.zeros((LANE,), jnp.float32)
            for ii in range(LANE):
                e = g * 16 + ii
                acc = qbuf[e, pl.ds(0, 16)] * kbuf[e, pl.ds(0, 16)]
                for c0 in range(16, C, 16):
                    acc = acc + qbuf[e, pl.ds(c0, 16)] * kbuf[e, pl.ds(c0, 16)]
                svec = jnp.where(iota == ii, _hsum16(acc, iota)[ii], svec)
            wbuf[pl.ds(g * 16, 16)] = jnp.exp(svec * inv)
            return 0
        lax.fori_loop(0, be // 16, score, 0)
        pltpu.sync_copy(wbuf, w_out.at[pl.ds(off, be)])
        for j2 in range(sub):
            pltpu.sync_copy(wbuf.at[pl.ds(j2 * 128, 128)], s_sh.at[dbuf.at[j2]], add=True)
        return 0
    lax.fori_loop(0, epw // be, blk, 0)
    plsc.subcore_barrier()

    def outl(i, _):
        rr = r0 + i * zc
        pltpu.sync_copy(s_sh.at[pl.ds(rr, zc)], zs1)
        pltpu.sync_copy(zs1, s_out.at[pl.ds(cid * n_pad + rr, zc)])
        return 0
    lax.fori_loop(0, rps // zc, outl, 0)


def _attn_score(qkv_view, src, dst, n_pad, C, e_pad, be):
    sub = be // 128
    zc = _zdiv(n_pad // NS, 512)
    kern = pl.kernel(
        functools.partial(_attn_score_body, n_pad=n_pad, C=C, e_pad=e_pad, be=be, zc=zc),
        out_type=(jax.ShapeDtypeStruct((e_pad,), jnp.float32),
                  jax.ShapeDtypeStruct((2 * n_pad,), jnp.float32)),
        mesh=plsc.VectorSubcoreMesh(core_axis_name="c", subcore_axis_name="s"),
        compiler_params=pltpu.CompilerParams(use_tc_tiling_on_sc=False),
        scratch_types=[
            pltpu.VMEM((sub, 128), jnp.int32),
            pltpu.VMEM((sub, 128), jnp.int32),
            pltpu.VMEM((sub, 128), jnp.int32),
            pltpu.VMEM((sub, 128), jnp.int32),
            pltpu.VMEM((be, C), jnp.float32),
            pltpu.VMEM((be, C), jnp.float32),
            pltpu.VMEM((be,), jnp.float32),
            pltpu.VMEM((zc,), jnp.float32),
            pltpu.VMEM_SHARED((n_pad,), jnp.float32),
            pltpu.SemaphoreType.DMA,
            pltpu.SemaphoreType.DMA,
        ],
    )
    return kern(qkv_view, src, dst)


def _attn_wv_body(v_view, srcr, dstr, w_hbm, agg_out,
                  sbuf, dbuf, vib, vbuf, wbuf, zrow,
                  agg_sh, semv,
                  *, n_pad, cc, e_pad, be, rpn, roff, zc):
    cid = lax.axis_index("c")
    sid = lax.axis_index("s")
    wid = sid * NC + cid
    rps = n_pad // NS
    r0 = sid * rps
    z16 = _zero16()

    def zr(r, _):
        for c0 in range(0, cc, 16):
            zrow[r, pl.ds(c0, 16)] = z16
        return 0
    lax.fori_loop(0, zc, zr, 0)

    def zcpy(i, _):
        pltpu.sync_copy(zrow, agg_sh.at[pl.ds(r0 + i * zc, zc)])
        return 0
    lax.fori_loop(0, rps // zc, zcpy, 0)
    plsc.subcore_barrier()
    epw = e_pad // NW
    base = wid * epw
    sub = be // 128
    iota = lax.iota(jnp.int32, LANE)

    def blk(j, _):
        off = base + j * be
        pltpu.sync_copy(w_hbm.at[pl.ds(off, be)], wbuf)

        def prep(j2, _):
            pltpu.sync_copy(srcr.at[pl.ds(off + j2 * 128, 128)], sbuf.at[j2])
            pltpu.sync_copy(dstr.at[pl.ds(off + j2 * 128, 128)], dbuf.at[j2])
            for gg in range(8):
                sl = pl.ds(gg * 16, 16)
                vib[j2, sl] = sbuf[j2, sl] * rpn + roff
            return 0
        lax.fori_loop(0, sub, prep, 0)
        cps = []
        for j2 in range(sub):
            cps.append(pltpu.async_copy(v_view.at[vib.at[j2]], vbuf.at[pl.ds(j2 * 128, 128)], semv))
        for c in cps:
            c.wait()

        def wv(g, _):
            w16 = wbuf[pl.ds(g * 16, 16)]
            for ii in range(LANE):
                e = g * 16 + ii
                we = w16[ii]
                for c0 in range(0, cc, 16):
                    vbuf[e, pl.ds(c0, 16)] = we * vbuf[e, pl.ds(c0, 16)]
            return 0
        lax.fori_loop(0, be // 16, wv, 0)
        for j2 in range(sub):
            pltpu.sync_copy(vbuf.at[pl.ds(j2 * 128, 128)], agg_sh.at[dbuf.at[j2]], add=True)
        return 0
    lax.fori_loop(0, epw // be, blk, 0)
    plsc.subcore_barrier()

    def outl(i, _):
        rr = r0 + i * zc
        pltpu.sync_copy(agg_sh.at[pl.ds(rr, zc)], zrow)
        pltpu.sync_copy(zrow, agg_out.at[cid, pl.ds(rr, zc)])
        return 0
    lax.fori_loop(0, rps // zc, outl, 0)


def _attn_wv(v_view, src, dst, w, n_pad, cc, e_pad, be, rpn, roff):
    sub = be // 128
    zc = _zdiv(n_pad // NS, 32)
    kern = pl.kernel(
        functools.partial(_attn_wv_body, n_pad=n_pad, cc=cc, e_pad=e_pad,
                          be=be, rpn=rpn, roff=roff, zc=zc),
        out_type=jax.ShapeDtypeStruct((2, n_pad, cc), jnp.float32),
        mesh=plsc.VectorSubcoreMesh(core_axis_name="c", subcore_axis_name="s"),
        compiler_params=pltpu.CompilerParams(use_tc_tiling_on_sc=False),
        scratch_types=[
            pltpu.VMEM((sub, 128), jnp.int32),
            pltpu.VMEM((sub, 128), jnp.int32),
            pltpu.VMEM((sub, 128), jnp.int32),
            pltpu.VMEM((be, cc), jnp.float32),
            pltpu.VMEM((be,), jnp.float32),
            pltpu.VMEM((zc, cc), jnp.float32),
            pltpu.VMEM_SHARED((n_pad, cc), jnp.float32),
            pltpu.SemaphoreType.DMA,
        ],
    )
    return kern(v_view, src, dst, w)


def _gather_body(table, idx, out, ibuf, rbuf, sem, *, nd_pad, chunk):
    cid = lax.axis_index("c")
    sid = lax.axis_index("s")
    wid = sid * NC + cid
    bpw = nd_pad // NW
    nchunk = bpw // chunk

    def go(j, _):
        base = wid * bpw + j * chunk
        pltpu.sync_copy(idx.at[pl.ds(base, chunk)], ibuf)
        pltpu.async_copy(table.at[ibuf], rbuf, sem).wait()
        pltpu.sync_copy(rbuf, out.at[pl.ds(base, chunk)])
        return 0
    lax.fori_loop(0, nchunk, go, 0)


def _gather_rows(table, idx, nd_pad, chunk):
    C = table.shape[1]
    kern = pl.kernel(
        functools.partial(_gather_body, nd_pad=nd_pad, chunk=chunk),
        out_type=jax.ShapeDtypeStruct((nd_pad, C), jnp.float32),
        mesh=plsc.VectorSubcoreMesh(core_axis_name="c", subcore_axis_name="s"),
        compiler_params=pltpu.CompilerParams(use_tc_tiling_on_sc=False),
        scratch_types=[
            pltpu.VMEM((chunk,), jnp.int32),
            pltpu.VMEM((chunk, C), jnp.float32),
            pltpu.SemaphoreType.DMA,
        ],
    )
    return kern(table, idx)


def _pad_rows(a, n_pad):
    n = a.shape[0]
    if n == n_pad:
        return a
    return jnp.concatenate(
        [a, jnp.zeros((n_pad - n,) + a.shape[1:], a.dtype)], axis=0)


def _pad_idx(idx, n_pad):
    n = idx.shape[0]
    if n == n_pad:
        return idx.astype(jnp.int32)
    return jnp.concatenate(
        [idx.astype(jnp.int32), jnp.zeros((n_pad - n,), jnp.int32)], axis=0)


def _tconv_layer(x, lp, src, dst, n, n_pad, e_pad, be, bl, ln, resid):
    C = x.shape[1]
    conv = lp['conv'] if ln else lp
    gam = lp['g'] if ln else jnp.ones((C,), jnp.float32)
    bet = lp['b'] if ln else jnp.zeros((C,), jnp.float32)
    out4 = _proj(x, gam, bet, conv, n_pad, bl, ln)
    if C <= 96:
        qkv_view = out4.reshape(n_pad * 4, C)
        s_part, agg_part = _attn_fused(qkv_view, src, dst, n_pad, C, e_pad, be)
        agg_parts = [agg_part]
    else:
        qkv_view = out4.reshape(n_pad * 4, C)
        w, s_part = _attn_score(qkv_view, src, dst, n_pad, C, e_pad, 256)
        cc = 32
        rpn = 4 * C // cc
        v_view = out4.reshape(n_pad * rpn, cc)
        agg_parts = []
        for ch in range(C // cc):
            roff = 2 * C // cc + ch
            agg_parts.append(
                _attn_wv(v_view, src, dst, w, n_pad, cc, e_pad, 128, rpn, roff))
    return _finalize(out4, s_part, agg_parts, x if resid else None, n_pad, bl, C)


def kernel(curr_pos, curr_motion, prev_motion, edge_index0, edge_index1,
           edge_index2, edge_index3, down0, down1, down2, up0, up1, up2, params):
    N0 = curr_pos.shape[0]
    N1 = down0.shape[0]
    N2 = down1.shape[0]
    N3 = down2.shape[0]
    T = prev_motion.shape[0] // N0
    N0p, N1p, N2p, N3p = _rup(N0, 512), _rup(N1, 512), _rup(N2, 512), _rup(N3, 512)
    BL0 = BL1 = BL2 = BL3 = 512

    def prep_edges(ei, n, e_align):
        E = ei.shape[1]
        Ep = _rup(E, e_align)
        src = _pad_idx(ei[0], Ep)
        dst = ei[1].astype(jnp.int32)
        if Ep != E:
            dst = jnp.concatenate([dst, jnp.full((Ep - E,), n, jnp.int32)])
        return src, dst, Ep

    src0, dst0, E0p = prep_edges(edge_index0, N0, NW * 512)
    src1, dst1, E1p = prep_edges(edge_index1, N1, NW * 256)
    src2, dst2, E2p = prep_edges(edge_index2, N2, NW * 128)
    src3, dst3, E3p = prep_edges(edge_index3, N3, NW * 128)

    pm = prev_motion.reshape(T, N0, 4)
    pm = jnp.concatenate([pm, jnp.zeros((T, N0p - N0, 4), jnp.float32)], axis=1)
    cp = _pad_rows(curr_pos, N0p)
    cm = _pad_rows(curr_motion, N0p)
    x = _lstm_enc(pm, cp, cm, params, N0p, BL0)

    L = params['layers']
    f0 = _tconv_layer(x, params['conv0'], src0, dst0, N0, N0p, E0p, 128, BL0,
                      ln=False, resid=False)
    f1 = _tconv_layer(f0, L['11'], src0, dst0, N0, N0p, E0p, 128, BL0, True, True)
    f1 = _tconv_layer(f1, L['12'], src0, dst0, N0, N0p, E0p, 128, BL0, True, True)

    f2 = _gather_rows(f1, _pad_idx(down0, N1p), N1p, 80)
    f2 = _tconv_layer(f2, L['21'], src1, dst1, N1, N1p, E1p, 256, BL1, True, True)
    f2 = _tconv_layer(f2, L['22'], src1, dst1, N1, N1p, E1p, 256, BL1, True, True)

    f3 = _gather_rows(f2, _pad_idx(down1, N2p), N2p, 112)
    f3 = _tconv_layer(f3, L['31'], src2, dst2, N2, N2p, E2p, 128, BL2, True, True)
    f3 = _tconv_layer(f3, L['32'], src2, dst2, N2, N2p, E2p, 128, BL2, True, True)

    f4 = _gather_rows(f3, _pad_idx(down2, N3p), N3p, 32)
    f4 = _tconv_layer(f4, L['41'], src3, dst3, N3, N3p, E3p, 512, BL3, True, True)
    f4 = _tconv_layer(f4, L['42'], src3, dst3, N3, N3p, E3p, 512, BL3, True, True)

    f5 = jnp.concatenate([_gather_rows(f4, _pad_idx(up2, N2p), N2p, 112), f3], axis=-1)
    f5 = _tconv_layer(f5, L['51'], src2, dst2, N2, N2p, E2p, 128, BL2, True, True)
    f5 = _tconv_layer(f5, L['52'], src2, dst2, N2, N2p, E2p, 128, BL2, True, True)

    f6 = jnp.concatenate([_gather_rows(f5, _pad_idx(up1, N1p), N1p, 80), f2], axis=-1)
    f6 = _tconv_layer(f6, L['61'], src1, dst1, N1, N1p, E1p, 128, BL1, True, True)
    f6 = _tconv_layer(f6, L['62'], src1, dst1, N1, N1p, E1p, 128, BL1, True, True)

    f7 = jnp.concatenate([_gather_rows(f6, _pad_idx(up0, N0p), N0p, 112), f1], axis=-1)
    f7 = _tconv_layer(f7, L['71'], src0, dst0, N0, N0p, E0p, 512, BL0, True, True)
    f7 = _tconv_layer(f7, L['72'], src0, dst0, N0, N0p, E0p, 512, BL0, True, True)

    out = _head(f7, params, N0p, BL0)
    return out[:N0, :4]


# speedup vs baseline: 10.1512x; 1.2151x over previous
import functools
import math

import jax
import jax.numpy as jnp
from jax import lax
from jax.experimental import pallas as pl
from jax.experimental.pallas import tpu as pltpu
from jax.experimental.pallas import tpu_sc as plsc

HID = 32
NC = 2
NS = 16
NW = NC * NS
LANE = 16


def _rup(x, m):
    return (x + m - 1) // m * m


def _lstm_enc_body(pm_ref, cp_ref, cm_ref, wih0_ref, whh0_ref, b0_ref,
                   wih1_ref, whh1_ref, b1_ref, seqw_ref, seqb_ref,
                   encw_ref, encb_ref, out_ref):
    T = pm_ref.shape[0]
    B = cp_ref.shape[0]
    h0 = jnp.zeros((B, HID), jnp.float32)
    c0 = jnp.zeros((B, HID), jnp.float32)
    h1 = jnp.zeros((B, HID), jnp.float32)
    c1 = jnp.zeros((B, HID), jnp.float32)

    def cell(xt, h, c, wih, whh, b):
        g = (jnp.dot(xt, wih, preferred_element_type=jnp.float32)
             + jnp.dot(h, whh, preferred_element_type=jnp.float32) + b)
        i = g[:, :HID]
        f = g[:, HID:2 * HID]
        gg = g[:, 2 * HID:3 * HID]
        o = g[:, 3 * HID:]
        i = 1.0 / (1.0 + jnp.exp(-i))
        f = 1.0 / (1.0 + jnp.exp(-f))
        gg = jnp.tanh(gg)
        o = 1.0 / (1.0 + jnp.exp(-o))
        c = f * c + i * gg
        h = o * jnp.tanh(c)
        return h, c

    for t in range(T):
        xt = pm_ref[t]
        h0, c0 = cell(xt, h0, c0, wih0_ref[...], whh0_ref[...], b0_ref[...])
        h1, c1 = cell(h0, h1, c1, wih1_ref[...], whh1_ref[...], b1_ref[...])
    seq_pred = jnp.dot(h1, seqw_ref[...], preferred_element_type=jnp.float32) + seqb_ref[...]
    encw = encw_ref[...]
    x = (jnp.dot(cp_ref[...], encw[:3], preferred_element_type=jnp.float32)
         + jnp.dot(seq_pred, encw[3:7], preferred_element_type=jnp.float32)
         + jnp.dot(cm_ref[...], encw[7:], preferred_element_type=jnp.float32)
         + encb_ref[...])
    out_ref[...] = x


def _lstm_enc(pm, cp, cm, p, n_pad, bl):
    T = pm.shape[0]
    grid = n_pad // bl
    full = lambda a: pl.BlockSpec(a.shape, lambda i: tuple(0 for _ in a.shape))
    l0, l1 = p['lstm'][0], p['lstm'][1]
    args = [
        pm, cp, cm,
        l0['Wih'].T, l0['Whh'].T, (l0['bih'] + l0['bhh']).reshape(1, -1),
        l1['Wih'].T, l1['Whh'].T, (l1['bih'] + l1['bhh']).reshape(1, -1),
        p['seq_W'], p['seq_b'].reshape(1, -1),
        p['enc_W'], p['enc_b'].reshape(1, -1),
    ]
    specs = [
        pl.BlockSpec((T, bl, 4), lambda i: (0, i, 0)),
        pl.BlockSpec((bl, 3), lambda i: (i, 0)),
        pl.BlockSpec((bl, 4), lambda i: (i, 0)),
    ] + [full(a) for a in args[3:]]
    return pl.pallas_call(
        _lstm_enc_body,
        grid=(grid,),
        in_specs=specs,
        out_specs=pl.BlockSpec((bl, HID), lambda i: (i, 0)),
        out_shape=jax.ShapeDtypeStruct((n_pad, HID), jnp.float32),
    )(*args)


def _proj_body(x_ref, g_ref, b_ref, w_ref, bias_ref, out_ref, *, ln):
    x = x_ref[...]
    if ln:
        m = jnp.mean(x, -1, keepdims=True)
        v = jnp.mean((x - m) ** 2, -1, keepdims=True)
        x = (x - m) / jnp.sqrt(v + 1e-5) * g_ref[...] + b_ref[...]
        x = jnp.maximum(x, 0.0)
    out_ref[...] = jnp.dot(x, w_ref[...], preferred_element_type=jnp.float32) + bias_ref[...]


def _proj(x, gam, bet, conv, n_pad, bl, ln):
    C = x.shape[1]
    wcat = jnp.concatenate([conv['Wq'], conv['Wk'], conv['Wv'], conv['Ws']], axis=1)
    bcat = jnp.concatenate([conv['bq'], conv['bk'], conv['bv'], conv['bs']]).reshape(1, -1)
    grid = n_pad // bl
    full = lambda a: pl.BlockSpec(a.shape, lambda i: tuple(0 for _ in a.shape))
    args = [x, gam.reshape(1, -1), bet.reshape(1, -1), wcat, bcat]
    specs = [pl.BlockSpec((bl, C), lambda i: (i, 0))] + [full(a) for a in args[1:]]
    return pl.pallas_call(
        functools.partial(_proj_body, ln=ln),
        grid=(grid,),
        in_specs=specs,
        out_specs=pl.BlockSpec((bl, 4 * C), lambda i: (i, 0)),
        out_shape=jax.ShapeDtypeStruct((n_pad, 4 * C), jnp.float32),
    )(*args)


def _fin_body(*refs, n_agg, resid):
    out_ref = refs[-1]
    x4 = refs[0][...]
    C = x4.shape[1] // 4
    sproj = x4[:, 3 * C:]
    s_ref = refs[1]
    aggs = refs[2:2 + n_agg]
    idx = 2 + n_agg
    s2 = s_ref[...]
    ssum = s2[0] + s2[1] + 1e-16
    parts = []
    for a in aggs:
        a2 = a[...]
        parts.append(a2[0] + a2[1])
    agg = jnp.concatenate(parts, axis=-1) if n_agg > 1 else parts[0]
    out = sproj + agg / ssum
    if resid:
        out = out + refs[idx][...]
    out_ref[...] = out


def _finalize(out4, s_part, agg_parts, xin, n_pad, bl, C):
    grid = n_pad // bl
    n_agg = len(agg_parts)
    cc = agg_parts[0].shape[2]
    s3 = s_part.reshape(2, n_pad, 1)
    args = [out4, s3] + list(agg_parts)
    specs = [
        pl.BlockSpec((bl, 4 * C), lambda i: (i, 0)),
        pl.BlockSpec((2, bl, 1), lambda i: (0, i, 0)),
    ] + [pl.BlockSpec((2, bl, cc), lambda i: (0, i, 0)) for _ in agg_parts]
    if xin is not None:
        args.append(xin)
        specs.append(pl.BlockSpec((bl, C), lambda i: (i, 0)))
    return pl.pallas_call(
        functools.partial(_fin_body, n_agg=n_agg, resid=xin is not None),
        grid=(grid,),
        in_specs=specs,
        out_specs=pl.BlockSpec((bl, C), lambda i: (i, 0)),
        out_shape=jax.ShapeDtypeStruct((n_pad, C), jnp.float32),
    )(*args)


def _head_body(x_ref, g_ref, b_ref, w_ref, bias_ref, out_ref):
    x = x_ref[...]
    m = jnp.mean(x, -1, keepdims=True)
    v = jnp.mean((x - m) ** 2, -1, keepdims=True)
    x = (x - m) / jnp.sqrt(v + 1e-5) * g_ref[...] + b_ref[...]
    x = jnp.maximum(x, 0.0)
    pred = jnp.dot(x, w_ref[...], preferred_element_type=jnp.float32) + bias_ref[...]
    lane = lax.broadcasted_iota(jnp.int32, pred.shape, 1)
    sp = jnp.where(pred > 20.0, pred, jnp.log1p(jnp.exp(jnp.minimum(pred, 20.0))))
    out_ref[...] = jnp.where(lane == 3, sp, pred)


def _head(x, p, n_pad, bl):
    C = x.shape[1]
    w8 = jnp.concatenate([p['lin_W'], jnp.zeros((C, 4), jnp.float32)], axis=1)
    b8 = jnp.concatenate([p['lin_b'], jnp.zeros((4,), jnp.float32)]).reshape(1, -1)
    grid = n_pad // bl
    full = lambda a: pl.BlockSpec(a.shape, lambda i: tuple(0 for _ in a.shape))
    args = [x, p['norm_g'].reshape(1, -1), p['norm_b'].reshape(1, -1), w8, b8]
    specs = [pl.BlockSpec((bl, C), lambda i: (i, 0))] + [full(a) for a in args[1:]]
    return pl.pallas_call(
        _head_body,
        grid=(grid,),
        in_specs=specs,
        out_specs=pl.BlockSpec((bl, 8), lambda i: (i, 0)),
        out_shape=jax.ShapeDtypeStruct((n_pad, 8), jnp.float32),
    )(*args)


def _zdiv(rps, cap):
    for d in range(min(rps, cap) // 16 * 16, 0, -16):
        if rps % d == 0:
            return d
    return 8


def _zero16():
    return jnp.zeros((LANE,), jnp.float32)


def _hsum16(acc, idx16):
    for step in (8, 4, 2, 1):
        acc = acc + jnp.take(acc, idx16 ^ step)
    return acc


def _attn_fused_body(qkv, srcr, dstr, s_out, agg_out,
                     sbuf, dbuf, qib, kib, vib, qbuf, kbuf, vbuf, wbuf,
                     zrow, zs1, s_sh, agg_sh,
                     semq0, semk0, semv0, semq1, semk1, semv1,
                     *, n_pad, C, e_pad, be, zc):
    cid = lax.axis_index("c")
    sid = lax.axis_index("s")
    wid = sid * NC + cid
    rps = n_pad // NS
    r0 = sid * rps
    z16 = _zero16()

    def zr(r, _):
        for c0 in range(0, C, 16):
            zrow[r, pl.ds(c0, 16)] = z16
        return 0
    lax.fori_loop(0, zc, zr, 0)

    def zs_(r, _):
        zs1[pl.ds(r * 16, 16)] = z16
        return 0
    lax.fori_loop(0, zc // 16, zs_, 0)

    def zcpy(i, _):
        rr = r0 + i * zc
        pltpu.sync_copy(zrow, agg_sh.at[pl.ds(rr, zc)])
        pltpu.sync_copy(zs1, s_sh.at[pl.ds(rr, zc)])
        return 0
    lax.fori_loop(0, rps // zc, zcpy, 0)
    plsc.subcore_barrier()
    epw = e_pad // NW
    base = wid * epw
    sub = be // 128
    nblk = epw // be
    inv = 1.0 / math.sqrt(C)
    iota = lax.iota(jnp.int32, LANE)
    sems = ((semq0, semk0, semv0), (semq1, semk1, semv1))

    def stage(j, slot):
        off = base + j * be

        def prep(j2, _):
            pltpu.sync_copy(srcr.at[pl.ds(off + j2 * 128, 128)], sbuf.at[slot, j2])
            pltpu.sync_copy(dstr.at[pl.ds(off + j2 * 128, 128)], dbuf.at[slot, j2])
            for gg in range(8):
                sl = pl.ds(gg * 16, 16)
                sv = sbuf[slot, j2, sl] * 4
                dv = dbuf[slot, j2, sl] * 4
                qib[slot, j2, sl] = dv
                kib[slot, j2, sl] = sv + 1
                vib[slot, j2, sl] = sv + 2
            return 0
        lax.fori_loop(0, sub, prep, 0)
        sq, sk, sv_ = sems[slot]
        for j2 in range(sub):
            pltpu.async_copy(qkv.at[qib.at[slot, j2]], qbuf.at[slot, pl.ds(j2 * 128, 128)], sq)
            pltpu.async_copy(qkv.at[kib.at[slot, j2]], kbuf.at[slot, pl.ds(j2 * 128, 128)], sk)
            pltpu.async_copy(qkv.at[vib.at[slot, j2]], vbuf.at[slot, pl.ds(j2 * 128, 128)], sv_)

    def drain(slot):
        sq, sk, sv_ = sems[slot]
        for j2 in range(sub):
            pltpu.make_async_copy(qkv.at[qib.at[slot, j2]], qbuf.at[slot, pl.ds(j2 * 128, 128)], sq).wait()
            pltpu.make_async_copy(qkv.at[kib.at[slot, j2]], kbuf.at[slot, pl.ds(j2 * 128, 128)], sk).wait()
            pltpu.make_async_copy(qkv.at[vib.at[slot, j2]], vbuf.at[slot, pl.ds(j2 * 128, 128)], sv_).wait()

    def work(j, slot):
        def score(g, _):
            svec = jnp.zeros((LANE,), jnp.float32)
            for ii in range(LANE):
                e = g * 16 + ii
                acc = qbuf[slot, e, pl.ds(0, 16)] * kbuf[slot, e, pl.ds(0, 16)]
                for c0 in range(16, C, 16):
                    acc = acc + qbuf[slot, e, pl.ds(c0, 16)] * kbuf[slot, e, pl.ds(c0, 16)]
                svec = jnp.where(iota == ii, _hsum16(acc, iota)[ii], svec)
            w = jnp.exp(svec * inv)
            wbuf[pl.ds(g * 16, 16)] = w
            for ii in range(LANE):
                e = g * 16 + ii
                we = w[ii]
                for c0 in range(0, C, 16):
                    vbuf[slot, e, pl.ds(c0, 16)] = we * vbuf[slot, e, pl.ds(c0, 16)]
            return 0
        lax.fori_loop(0, be // 16, score, 0)
        for j2 in range(sub):
            pltpu.sync_copy(wbuf.at[pl.ds(j2 * 128, 128)], s_sh.at[dbuf.at[slot, j2]], add=True)
            pltpu.sync_copy(vbuf.at[slot, pl.ds(j2 * 128, 128)], agg_sh.at[dbuf.at[slot, j2]], add=True)

    stage(0, 0)

    def blk2(jj, _):
        for b in range(2):
            j = jj * 2 + b

            @pl.when(j + 1 < nblk)
            def _():
                stage(j + 1, 1 - b)
            drain(b)
            work(j, b)
        return 0
    lax.fori_loop(0, nblk // 2, blk2, 0)
    if nblk % 2:
        drain((nblk - 1) % 2)
        work(nblk - 1, (nblk - 1) % 2)
    plsc.subcore_barrier()

    def outl(i, _):
        rr = r0 + i * zc
        pltpu.sync_copy(s_sh.at[pl.ds(rr, zc)], zs1)
        pltpu.sync_copy(zs1, s_out.at[pl.ds(cid * n_pad + rr, zc)])
        pltpu.sync_copy(agg_sh.at[pl.ds(rr, zc)], zrow)
        pltpu.sync_copy(zrow, agg_out.at[cid, pl.ds(rr, zc)])
        return 0
    lax.fori_loop(0, rps // zc, outl, 0)


def _attn_fused(qkv_view, src, dst, n_pad, C, e_pad, be):
    sub = be // 128
    zc = _zdiv(n_pad // NS, 32)
    kern = pl.kernel(
        functools.partial(_attn_fused_body, n_pad=n_pad, C=C, e_pad=e_pad, be=be, zc=zc),
        out_type=(jax.ShapeDtypeStruct((2 * n_pad,), jnp.float32),
                  jax.ShapeDtypeStruct((2, n_pad, C), jnp.float32)),
        mesh=plsc.VectorSubcoreMesh(core_axis_name="c", subcore_axis_name="s"),
        compiler_params=pltpu.CompilerParams(use_tc_tiling_on_sc=False),
        scratch_types=[
            pltpu.VMEM((2, sub, 128), jnp.int32),
            pltpu.VMEM((2, sub, 128), jnp.int32),
            pltpu.VMEM((2, sub, 128), jnp.int32),
            pltpu.VMEM((2, sub, 128), jnp.int32),
            pltpu.VMEM((2, sub, 128), jnp.int32),
            pltpu.VMEM((2, be, C), jnp.float32),
            pltpu.VMEM((2, be, C), jnp.float32),
            pltpu.VMEM((2, be, C), jnp.float32),
            pltpu.VMEM((be,), jnp.float32),
            pltpu.VMEM((zc, C), jnp.float32),
            pltpu.VMEM((zc,), jnp.float32),
            pltpu.VMEM_SHARED((n_pad,), jnp.float32),
            pltpu.VMEM_SHARED((n_pad, C), jnp.float32),
        ] + [pltpu.SemaphoreType.DMA] * 6,
    )
    return kern(qkv_view, src, dst)


def _attn_score_body(qkv, srcr, dstr, w_out, s_out,
                     sbuf, dbuf, qib, kib, qbuf, kbuf, wbuf, zs1,
                     s_sh, semq0, semk0, semq1, semk1,
                     *, n_pad, C, e_pad, be, zc):
    cid = lax.axis_index("c")
    sid = lax.axis_index("s")
    wid = sid * NC + cid
    rps = n_pad // NS
    r0 = sid * rps
    z16 = _zero16()

    def zs_(r, _):
        zs1[pl.ds(r * 16, 16)] = z16
        return 0
    lax.fori_loop(0, zc // 16, zs_, 0)

    def zcpy(i, _):
        pltpu.sync_copy(zs1, s_sh.at[pl.ds(r0 + i * zc, zc)])
        return 0
    lax.fori_loop(0, rps // zc, zcpy, 0)
    plsc.subcore_barrier()
    epw = e_pad // NW
    base = wid * epw
    sub = be // 128
    nblk = epw // be
    inv = 1.0 / math.sqrt(C)
    iota = lax.iota(jnp.int32, LANE)
    sems = ((semq0, semk0), (semq1, semk1))

    def stage(j, slot):
        off = base + j * be

        def prep(j2, _):
            pltpu.sync_copy(srcr.at[pl.ds(off + j2 * 128, 128)], sbuf.at[slot, j2])
            pltpu.sync_copy(dstr.at[pl.ds(off + j2 * 128, 128)], dbuf.at[slot, j2])
            for gg in range(8):
                sl = pl.ds(gg * 16, 16)
                qib[slot, j2, sl] = dbuf[slot, j2, sl] * 4
                kib[slot, j2, sl] = sbuf[slot, j2, sl] * 4 + 1
            return 0
        lax.fori_loop(0, sub, prep, 0)
        sq, sk = sems[slot]
        for j2 in range(sub):
            pltpu.async_copy(qkv.at[qib.at[slot, j2]], qbuf.at[slot, pl.ds(j2 * 128, 128)], sq)
            pltpu.async_copy(qkv.at[kib.at[slot, j2]], kbuf.at[slot, pl.ds(j2 * 128, 128)], sk)

    def drain(slot):
        sq, sk = sems[slot]
        for j2 in range(sub):
            pltpu.make_async_copy(qkv.at[qib.at[slot, j2]], qbuf.at[slot, pl.ds(j2 * 128, 128)], sq).wait()
            pltpu.make_async_copy(qkv.at[kib.at[slot, j2]], kbuf.at[slot, pl.ds(j2 * 128, 128)], sk).wait()

    def work(j, slot):
        off = base + j * be

        def score(g, _):
            svec = jnp.zeros((LANE,), jnp.float32)
            for ii in range(LANE):
                e = g * 16 + ii
                acc = qbuf[slot, e, pl.ds(0, 16)] * kbuf[slot, e, pl.ds(0, 16)]
                for c0 in range(16, C, 16):
                    acc = acc + qbuf[slot, e, pl.ds(c0, 16)] * kbuf[slot, e, pl.ds(c0, 16)]
                svec = jnp.where(iota == ii, _hsum16(acc, iota)[ii], svec)
            wbuf[pl.ds(g * 16, 16)] = jnp.exp(svec * inv)
            return 0
        lax.fori_loop(0, be // 16, score, 0)
        pltpu.sync_copy(wbuf, w_out.at[pl.ds(off, be)])
        for j2 in range(sub):
            pltpu.sync_copy(wbuf.at[pl.ds(j2 * 128, 128)], s_sh.at[dbuf.at[slot, j2]], add=True)

    stage(0, 0)

    def blk2(jj, _):
        for b in range(2):
            j = jj * 2 + b

            @pl.when(j + 1 < nblk)
            def _():
                stage(j + 1, 1 - b)
            drain(b)
            work(j, b)
        return 0
    lax.fori_loop(0, nblk // 2, blk2, 0)
    if nblk % 2:
        drain((nblk - 1) % 2)
        work(nblk - 1, (nblk - 1) % 2)
    plsc.subcore_barrier()

    def outl(i, _):
        rr = r0 + i * zc
        pltpu.sync_copy(s_sh.at[pl.ds(rr, zc)], zs1)
        pltpu.sync_copy(zs1, s_out.at[pl.ds(cid * n_pad + rr, zc)])
        return 0
    lax.fori_loop(0, rps // zc, outl, 0)


def _attn_score(qkv_view, src, dst, n_pad, C, e_pad, be):
    sub = be // 128
    zc = _zdiv(n_pad // NS, 448)
    kern = pl.kernel(
        functools.partial(_attn_score_body, n_pad=n_pad, C=C, e_pad=e_pad, be=be, zc=zc),
        out_type=(jax.ShapeDtypeStruct((e_pad,), jnp.float32),
                  jax.ShapeDtypeStruct((2 * n_pad,), jnp.float32)),
        mesh=plsc.VectorSubcoreMesh(core_axis_name="c", subcore_axis_name="s"),
        compiler_params=pltpu.CompilerParams(use_tc_tiling_on_sc=False),
        scratch_types=[
            pltpu.VMEM((2, sub, 128), jnp.int32),
            pltpu.VMEM((2, sub, 128), jnp.int32),
            pltpu.VMEM((2, sub, 128), jnp.int32),
            pltpu.VMEM((2, sub, 128), jnp.int32),
            pltpu.VMEM((2, be, C), jnp.float32),
            pltpu.VMEM((2, be, C), jnp.float32),
            pltpu.VMEM((be,), jnp.float32),
            pltpu.VMEM((zc,), jnp.float32),
            pltpu.VMEM_SHARED((n_pad,), jnp.float32),
        ] + [pltpu.SemaphoreType.DMA] * 4,
    )
    return kern(qkv_view, src, dst)


def _attn_wv_body(v_view, srcr, dstr, w_hbm, agg_out,
                  sbuf, dbuf, vib, vbuf, wbuf, zrow,
                  agg_sh, semv0, semv1,
                  *, n_pad, cc, e_pad, be, rpn, roff, zc):
    cid = lax.axis_index("c")
    sid = lax.axis_index("s")
    wid = sid * NC + cid
    rps = n_pad // NS
    r0 = sid * rps
    z16 = _zero16()

    def zr(r, _):
        for c0 in range(0, cc, 16):
            zrow[r, pl.ds(c0, 16)] = z16
        return 0
    lax.fori_loop(0, zc, zr, 0)

    def zcpy(i, _):
        pltpu.sync_copy(zrow, agg_sh.at[pl.ds(r0 + i * zc, zc)])
        return 0
    lax.fori_loop(0, rps // zc, zcpy, 0)
    plsc.subcore_barrier()
    epw = e_pad // NW
    base = wid * epw
    sub = be // 128
    nblk = epw // be
    sems = (semv0, semv1)

    def stage(j, slot):
        off = base + j * be

        def prep(j2, _):
            pltpu.sync_copy(srcr.at[pl.ds(off + j2 * 128, 128)], sbuf.at[slot, j2])
            pltpu.sync_copy(dstr.at[pl.ds(off + j2 * 128, 128)], dbuf.at[slot, j2])
            for gg in range(8):
                sl = pl.ds(gg * 16, 16)
                vib[slot, j2, sl] = sbuf[slot, j2, sl] * rpn + roff
            return 0
        lax.fori_loop(0, sub, prep, 0)
        for j2 in range(sub):
            pltpu.async_copy(v_view.at[vib.at[slot, j2]], vbuf.at[slot, pl.ds(j2 * 128, 128)], sems[slot])

    def drain(slot):
        for j2 in range(sub):
            pltpu.make_async_copy(v_view.at[vib.at[slot, j2]], vbuf.at[slot, pl.ds(j2 * 128, 128)], sems[slot]).wait()

    def work(j, slot):
        off = base + j * be
        pltpu.sync_copy(w_hbm.at[pl.ds(off, be)], wbuf)

        def wv(g, _):
            w16 = wbuf[pl.ds(g * 16, 16)]
            for ii in range(LANE):
                e = g * 16 + ii
                we = w16[ii]
                for c0 in range(0, cc, 16):
                    vbuf[slot, e, pl.ds(c0, 16)] = we * vbuf[slot, e, pl.ds(c0, 16)]
            return 0
        lax.fori_loop(0, be // 16, wv, 0)
        for j2 in range(sub):
            pltpu.sync_copy(vbuf.at[slot, pl.ds(j2 * 128, 128)], agg_sh.at[dbuf.at[slot, j2]], add=True)

    stage(0, 0)

    def blk2(jj, _):
        for b in range(2):
            j = jj * 2 + b

            @pl.when(j + 1 < nblk)
            def _():
                stage(j + 1, 1 - b)
            drain(b)
            work(j, b)
        return 0
    lax.fori_loop(0, nblk // 2, blk2, 0)
    if nblk % 2:
        drain((nblk - 1) % 2)
        work(nblk - 1, (nblk - 1) % 2)
    plsc.subcore_barrier()

    def outl(i, _):
        rr = r0 + i * zc
        pltpu.sync_copy(agg_sh.at[pl.ds(rr, zc)], zrow)
        pltpu.sync_copy(zrow, agg_out.at[cid, pl.ds(rr, zc)])
        return 0
    lax.fori_loop(0, rps // zc, outl, 0)


def _attn_wv(v_view, src, dst, w, n_pad, cc, e_pad, be, rpn, roff):
    sub = be // 128
    zc = _zdiv(n_pad // NS, 32)
    kern = pl.kernel(
        functools.partial(_attn_wv_body, n_pad=n_pad, cc=cc, e_pad=e_pad,
                          be=be, rpn=rpn, roff=roff, zc=zc),
        out_type=jax.ShapeDtypeStruct((2, n_pad, cc), jnp.float32),
        mesh=plsc.VectorSubcoreMesh(core_axis_name="c", subcore_axis_name="s"),
        compiler_params=pltpu.CompilerParams(use_tc_tiling_on_sc=False),
        scratch_types=[
            pltpu.VMEM((2, sub, 128), jnp.int32),
            pltpu.VMEM((2, sub, 128), jnp.int32),
            pltpu.VMEM((2, sub, 128), jnp.int32),
            pltpu.VMEM((2, be, cc), jnp.float32),
            pltpu.VMEM((be,), jnp.float32),
            pltpu.VMEM((zc, cc), jnp.float32),
            pltpu.VMEM_SHARED((n_pad, cc), jnp.float32),
        ] + [pltpu.SemaphoreType.DMA] * 2,
    )
    return kern(v_view, src, dst, w)


def _gather_body(table, idx, out, ibuf, rbuf, sem, *, nd_pad, chunk):
    cid = lax.axis_index("c")
    sid = lax.axis_index("s")
    wid = sid * NC + cid
    bpw = nd_pad // NW
    nchunk = bpw // chunk

    def go(j, _):
        base = wid * bpw + j * chunk
        pltpu.sync_copy(idx.at[pl.ds(base, chunk)], ibuf)
        pltpu.async_copy(table.at[ibuf], rbuf, sem).wait()
        pltpu.sync_copy(rbuf, out.at[pl.ds(base, chunk)])
        return 0
    lax.fori_loop(0, nchunk, go, 0)


def _gather_rows(table, idx, nd_pad, chunk):
    C = table.shape[1]
    kern = pl.kernel(
        functools.partial(_gather_body, nd_pad=nd_pad, chunk=chunk),
        out_type=jax.ShapeDtypeStruct((nd_pad, C), jnp.float32),
        mesh=plsc.VectorSubcoreMesh(core_axis_name="c", subcore_axis_name="s"),
        compiler_params=pltpu.CompilerParams(use_tc_tiling_on_sc=False),
        scratch_types=[
            pltpu.VMEM((chunk,), jnp.int32),
            pltpu.VMEM((chunk, C), jnp.float32),
            pltpu.SemaphoreType.DMA,
        ],
    )
    return kern(table, idx)


def _pad_rows(a, n_pad):
    n = a.shape[0]
    if n == n_pad:
        return a
    return jnp.concatenate(
        [a, jnp.zeros((n_pad - n,) + a.shape[1:], a.dtype)], axis=0)


def _pad_idx(idx, n_pad):
    n = idx.shape[0]
    if n == n_pad:
        return idx.astype(jnp.int32)
    return jnp.concatenate(
        [idx.astype(jnp.int32), jnp.zeros((n_pad - n,), jnp.int32)], axis=0)


def _tconv_layer(x, lp, src, dst, n, n_pad, e_pad, be, bl, ln, resid):
    C = x.shape[1]
    conv = lp['conv'] if ln else lp
    gam = lp['g'] if ln else jnp.ones((C,), jnp.float32)
    bet = lp['b'] if ln else jnp.zeros((C,), jnp.float32)
    out4 = _proj(x, gam, bet, conv, n_pad, bl, ln)
    if C <= 64:
        qkv_view = out4.reshape(n_pad * 4, C)
        s_part, agg_part = _attn_fused(qkv_view, src, dst, n_pad, C, e_pad, be)
        agg_parts = [agg_part]
    else:
        qkv_view = out4.reshape(n_pad * 4, C)
        w, s_part = _attn_score(qkv_view, src, dst, n_pad, C, e_pad, 128)
        cc = 32
        rpn = 4 * C // cc
        v_view = out4.reshape(n_pad * rpn, cc)
        agg_parts = []
        for ch in range(C // cc):
            roff = 2 * C // cc + ch
            agg_parts.append(
                _attn_wv(v_view, src, dst, w, n_pad, cc, e_pad, 128, rpn, roff))
    return _finalize(out4, s_part, agg_parts, x if resid else None, n_pad, bl, C)


def kernel(curr_pos, curr_motion, prev_motion, edge_index0, edge_index1,
           edge_index2, edge_index3, down0, down1, down2, up0, up1, up2, params):
    N0 = curr_pos.shape[0]
    N1 = down0.shape[0]
    N2 = down1.shape[0]
    N3 = down2.shape[0]
    T = prev_motion.shape[0] // N0
    N0p, N1p, N2p, N3p = _rup(N0, 512), _rup(N1, 512), _rup(N2, 512), _rup(N3, 512)
    BL0 = BL1 = BL2 = BL3 = 512

    def prep_edges(ei, n, e_align):
        E = ei.shape[1]
        Ep = _rup(E, e_align)
        src = _pad_idx(ei[0], Ep)
        dst = ei[1].astype(jnp.int32)
        if Ep != E:
            dst = jnp.concatenate([dst, jnp.full((Ep - E,), n, jnp.int32)])
        return src, dst, Ep

    src0, dst0, E0p = prep_edges(edge_index0, N0, NW * 512)
    src1, dst1, E1p = prep_edges(edge_index1, N1, NW * 256)
    src2, dst2, E2p = prep_edges(edge_index2, N2, NW * 128)
    src3, dst3, E3p = prep_edges(edge_index3, N3, NW * 128)

    pm = prev_motion.reshape(T, N0, 4)
    pm = jnp.concatenate([pm, jnp.zeros((T, N0p - N0, 4), jnp.float32)], axis=1)
    cp = _pad_rows(curr_pos, N0p)
    cm = _pad_rows(curr_motion, N0p)
    x = _lstm_enc(pm, cp, cm, params, N0p, BL0)

    L = params['layers']
    f0 = _tconv_layer(x, params['conv0'], src0, dst0, N0, N0p, E0p, 128, BL0,
                      ln=False, resid=False)
    f1 = _tconv_layer(f0, L['11'], src0, dst0, N0, N0p, E0p, 128, BL0, True, True)
    f1 = _tconv_layer(f1, L['12'], src0, dst0, N0, N0p, E0p, 128, BL0, True, True)

    f2 = _gather_rows(f1, _pad_idx(down0, N1p), N1p, 80)
    f2 = _tconv_layer(f2, L['21'], src1, dst1, N1, N1p, E1p, 256, BL1, True, True)
    f2 = _tconv_layer(f2, L['22'], src1, dst1, N1, N1p, E1p, 256, BL1, True, True)

    f3 = _gather_rows(f2, _pad_idx(down1, N2p), N2p, 112)
    f3 = _tconv_layer(f3, L['31'], src2, dst2, N2, N2p, E2p, 128, BL2, True, True)
    f3 = _tconv_layer(f3, L['32'], src2, dst2, N2, N2p, E2p, 128, BL2, True, True)

    f4 = _gather_rows(f3, _pad_idx(down2, N3p), N3p, 32)
    f4 = _tconv_layer(f4, L['41'], src3, dst3, N3, N3p, E3p, 512, BL3, True, True)
    f4 = _tconv_layer(f4, L['42'], src3, dst3, N3, N3p, E3p, 512, BL3, True, True)

    f5 = jnp.concatenate([_gather_rows(f4, _pad_idx(up2, N2p), N2p, 112), f3], axis=-1)
    f5 = _tconv_layer(f5, L['51'], src2, dst2, N2, N2p, E2p, 128, BL2, True, True)
    f5 = _tconv_layer(f5, L['52'], src2, dst2, N2, N2p, E2p, 128, BL2, True, True)

    f6 = jnp.concatenate([_gather_rows(f5, _pad_idx(up1, N1p), N1p, 80), f2], axis=-1)
    f6 = _tconv_layer(f6, L['61'], src1, dst1, N1, N1p, E1p, 128, BL1, True, True)
    f6 = _tconv_layer(f6, L['62'], src1, dst1, N1, N1p, E1p, 128, BL1, True, True)

    f7 = jnp.concatenate([_gather_rows(f6, _pad_idx(up0, N0p), N0p, 112), f1], axis=-1)
    f7 = _tconv_layer(f7, L['71'], src0, dst0, N0, N0p, E0p, 512, BL0, True, True)
    f7 = _tconv_layer(f7, L['72'], src0, dst0, N0, N0p, E0p, 512, BL0, True, True)

    out = _head(f7, params, N0p, BL0)
    return out[:N0, :4]
